# baseline (speedup 1.0000x reference)
import functools

import jax
import jax.numpy as jnp
from jax import lax
from jax.experimental import pallas as pl
from jax.experimental.pallas import tpu as pltpu
from jax.experimental.pallas import tpu_sc as plsc

N_NODES = 10000
N_NODES_P = 10240
N_EDGES = 320000
PER_TILE = N_EDGES // 32
SL = N_NODES_P // 16

_MESH = plsc.VectorSubcoreMesh(core_axis_name="c", subcore_axis_name="s")


@functools.partial(
    pl.kernel,
    mesh=_MESH,
    compiler_params=pltpu.CompilerParams(
        needs_layout_passes=False, skip_device_barrier=True),
    out_type=jax.ShapeDtypeStruct((2, N_NODES_P), jnp.float32),
    scratch_types=[
        pltpu.VMEM((2, 10240), jnp.int32),
        pltpu.VMEM((N_NODES_P,), jnp.float32),
        pltpu.VMEM((16, SL), jnp.float32),
        pltpu.VMEM((SL,), jnp.float32),
        pltpu.VMEM_SHARED((16, N_NODES_P), jnp.float32),
        pltpu.SemaphoreType.DMA,
    ],
)
def _sc_degree_hist(edges_hbm, out_hbm, idx_v, hist_v, comb_v, acc_v, hists_sh, sem):
    c = lax.axis_index("c")
    s = lax.axis_index("s")
    w = c * 16 + s
    start = w * PER_TILE
    aligned = jnp.minimum((start // 128) * 128, N_EDGES - 10240)
    off = start - aligned
    cp = pltpu.async_copy(
        edges_hbm.at[:, pl.ds(aligned, 10240)], idx_v, sem)

    zeros16 = jnp.zeros((16,), jnp.float32)
    ones16 = jnp.ones((16,), jnp.float32)

    @plsc.parallel_loop(0, N_NODES_P // 16, unroll=8)
    def _(j):
        hist_v[pl.ds(j * 16, 16)] = zeros16

    cp.wait()

    @plsc.parallel_loop(0, PER_TILE // 16, unroll=16)
    def _(j):
        idx16 = idx_v[1, pl.ds(off + j * 16, 16)]
        plsc.addupdate_scatter(hist_v, [idx16], ones16)

    pltpu.sync_copy(hist_v, hists_sh.at[s])
    plsc.subcore_barrier()
    pltpu.sync_copy(hists_sh.at[:, pl.ds(s * SL, SL)], comb_v)

    @plsc.parallel_loop(0, SL // 16, unroll=4)
    def _(i):
        acc = comb_v[0, pl.ds(i * 16, 16)]
        for r in range(1, 16):
            acc = acc + comb_v[r, pl.ds(i * 16, 16)]
        acc_v[pl.ds(i * 16, 16)] = acc

    pltpu.sync_copy(acc_v, out_hbm.at[c, pl.ds(s * SL, SL)])


_TC_CH = 1024
_TC_NCH = 10


def _tc_linear_body(part_ref, w_ref, b_ref, out_ref, acc_ref, sem):
    s_row = jnp.sum(w_ref[...], axis=1)
    s_rep = jnp.broadcast_to(s_row[None, :], (2, 128))
    bias = b_ref[...]
    cps = []
    for k in range(_TC_NCH):
        deg2 = part_ref[:, k * _TC_CH:(k + 1) * _TC_CH]
        acc = lax.dot_general(
            deg2, s_rep, (((0,), (0,)), ((), ())),
            preferred_element_type=jnp.float32,
        ) + bias
        acc_ref[k] = acc
        rows = min(N_NODES - k * _TC_CH, _TC_CH)
        cps.append(pltpu.async_copy(
            acc_ref.at[k, pl.ds(0, rows)],
            out_ref.at[pl.ds(k * _TC_CH, rows)], sem))
    for cp in cps:
        cp.wait()


def kernel(edge_index, W, b):
    part = _sc_degree_hist(edge_index)

    out = pl.pallas_call(
        _tc_linear_body,
        out_shape=jax.ShapeDtypeStruct((N_NODES, 128), jnp.float32),
        out_specs=pl.BlockSpec(memory_space=pl.ANY),
        scratch_shapes=[
            pltpu.VMEM((_TC_NCH, _TC_CH, 128), jnp.float32),
            pltpu.SemaphoreType.DMA,
        ],
    )(part, W, b.reshape(1, 128))
    return out


# speedup vs baseline: 1.0076x; 1.0076x over previous
import functools

import jax
import jax.numpy as jnp
from jax import lax
from jax.experimental import pallas as pl
from jax.experimental.pallas import tpu as pltpu
from jax.experimental.pallas import tpu_sc as plsc

N_NODES = 10000
N_NODES_P = 10240
N_EDGES = 320000
PER_TILE = N_EDGES // 32
SL = N_NODES_P // 16

_MESH = plsc.VectorSubcoreMesh(core_axis_name="c", subcore_axis_name="s")


@functools.partial(
    pl.kernel,
    mesh=_MESH,
    compiler_params=pltpu.CompilerParams(
        needs_layout_passes=False, skip_device_barrier=True),
    out_type=jax.ShapeDtypeStruct((2, N_NODES_P), jnp.float32),
    scratch_types=[
        pltpu.VMEM((2, 10240), jnp.int32),
        pltpu.VMEM((N_NODES_P,), jnp.float32),
        pltpu.VMEM((16, SL), jnp.float32),
        pltpu.VMEM((SL,), jnp.float32),
        pltpu.VMEM_SHARED((16, N_NODES_P), jnp.float32),
        pltpu.SemaphoreType.DMA,
    ],
)
def _sc_degree_hist(edges_hbm, out_hbm, idx_v, hist_v, comb_v, acc_v, hists_sh, sem):
    c = lax.axis_index("c")
    s = lax.axis_index("s")
    w = c * 16 + s
    start = w * PER_TILE
    aligned = jnp.minimum((start // 128) * 128, N_EDGES - 10240)
    off = start - aligned
    cp = pltpu.async_copy(
        edges_hbm.at[:, pl.ds(aligned, 10240)], idx_v, sem)

    zeros16 = jnp.zeros((16,), jnp.float32)
    ones16 = jnp.ones((16,), jnp.float32)

    @plsc.parallel_loop(0, N_NODES_P // 16, unroll=8)
    def _(j):
        hist_v[pl.ds(j * 16, 16)] = zeros16

    cp.wait()

    @plsc.parallel_loop(0, PER_TILE // 16, unroll=8)
    def _(j):
        idx16 = idx_v[1, pl.ds(off + j * 16, 16)]
        plsc.addupdate_scatter(hist_v, [idx16], ones16)

    pltpu.sync_copy(hist_v, hists_sh.at[s])
    plsc.subcore_barrier()
    pltpu.sync_copy(hists_sh.at[:, pl.ds(s * SL, SL)], comb_v)

    @plsc.parallel_loop(0, SL // 16, unroll=4)
    def _(i):
        acc = comb_v[0, pl.ds(i * 16, 16)]
        for r in range(1, 16):
            acc = acc + comb_v[r, pl.ds(i * 16, 16)]
        acc_v[pl.ds(i * 16, 16)] = acc

    pltpu.sync_copy(acc_v, out_hbm.at[c, pl.ds(s * SL, SL)])


_TC_CH = 1024
_TC_NCH = 10


def _tc_linear_body(part_ref, w_ref, b_ref, out_ref, acc_ref, sem):
    s_row = jnp.sum(w_ref[...], axis=1)
    s_rep = jnp.broadcast_to(s_row[None, :], (2, 128))
    bias = b_ref[...]
    cps = []
    for k in range(_TC_NCH):
        deg2 = part_ref[:, k * _TC_CH:(k + 1) * _TC_CH]
        acc = lax.dot_general(
            deg2, s_rep, (((0,), (0,)), ((), ())),
            preferred_element_type=jnp.float32,
        ) + bias
        acc_ref[k] = acc
        rows = min(N_NODES - k * _TC_CH, _TC_CH)
        cps.append(pltpu.async_copy(
            acc_ref.at[k, pl.ds(0, rows)],
            out_ref.at[pl.ds(k * _TC_CH, rows)], sem))
    for cp in cps:
        cp.wait()


def kernel(edge_index, W, b):
    part = _sc_degree_hist(edge_index)

    out = pl.pallas_call(
        _tc_linear_body,
        out_shape=jax.ShapeDtypeStruct((N_NODES, 128), jnp.float32),
        out_specs=pl.BlockSpec(memory_space=pl.ANY),
        scratch_shapes=[
            pltpu.VMEM((_TC_NCH, _TC_CH, 128), jnp.float32),
            pltpu.SemaphoreType.DMA,
        ],
    )(part, W, b.reshape(1, 128))
    return out
